# Initial kernel scaffold; baseline (speedup 1.0000x reference)
import jax
import jax.numpy as jnp
from jax.experimental import pallas as pl


def kernel(x, W1, g1, b1, W2, g2, b2, W2m, g2m, b2m, W3, g3, b3, W4, g4, b4, W5, g5, b5, L1, g6, b6, L2, bl2, g7, b7, L3, bl3):
    raise NotImplementedError("write your pallas kernel here")


# speedup vs baseline: 9.7741x; 9.7741x over previous
import functools

import jax
import jax.numpy as jnp
from jax import lax
from jax.experimental import pallas as pl
from jax.experimental.pallas import tpu as pltpu
from jax.experimental.pallas import tpu_sc as plsc

EPS = 1e-5
KNN = 20
B = 8
N = 1024
P = B * N
NEG = -3.0e38

NW = 32
PPW = P // NW
CP = 4
CH = PPW // CP

NT = 128
NB = N // NT


def _full_spec(shape):
  return pl.BlockSpec(shape, lambda *a: (0,) * len(shape))


def _front_body(feat, idx_ref):
  b = pl.program_id(0)
  F = feat[0]
  xx = jnp.sum(F * F, axis=1)
  G = lax.dot_general(F, F, (((1,), (1,)), ((), ())),
                      preferred_element_type=jnp.float32)
  d = 2.0 * G - xx[:, None] - xx[None, :]
  iota2 = lax.broadcasted_iota(jnp.int32, (N, N), 1)
  cols = []
  for _ in range(KNN):
    rm = jnp.max(d, axis=1, keepdims=True)
    cand = jnp.where(d == rm, iota2, N)
    sel = jnp.min(cand, axis=1, keepdims=True)
    cols.append(sel)
    d = jnp.where(iota2 == sel, NEG, d)
  idx_ref[0] = jnp.concatenate(cols, axis=1) + b * N


def _front(feat, interpret=False):
  C = feat.shape[-1]
  return pl.pallas_call(
      _front_body,
      grid=(B,),
      in_specs=[pl.BlockSpec((1, N, C), lambda b: (b, 0, 0))],
      out_specs=pl.BlockSpec((1, N, KNN), lambda b: (b, 0, 0)),
      out_shape=jax.ShapeDtypeStruct((B, N, KNN), jnp.int32),
      interpret=interpret,
  )(feat)


def _sc_gather(feat_pad, idx_flat, C_pad):
  mesh = plsc.VectorSubcoreMesh(core_axis_name="c", subcore_axis_name="s")

  @functools.partial(
      pl.kernel,
      mesh=mesh,
      out_type=jax.ShapeDtypeStruct((P * KNN, C_pad), jnp.float32),
      scratch_types=[
          pltpu.VMEM((CP * KNN,), jnp.int32),
          pltpu.VMEM((CP * KNN, C_pad), jnp.float32),
          pltpu.SemaphoreType.DMA,
      ],
  )
  def k(feat_hbm, idx_hbm, out_hbm, idx_v, rows_v, sem):
    w = lax.axis_index("s") * 2 + lax.axis_index("c")

    def chunk(c, carry):
      off = w * (PPW * KNN) + c * (CP * KNN)
      pltpu.sync_copy(idx_hbm.at[pl.ds(off, CP * KNN)], idx_v)
      pltpu.async_copy(feat_hbm.at[idx_v], rows_v, sem).wait()
      pltpu.sync_copy(rows_v, out_hbm.at[pl.ds(off, CP * KNN)])
      return carry

    lax.fori_loop(0, CH, chunk, 0, unroll=False)

  return k(feat_pad, idx_flat)


def _conv_body(C, dup, nbr, feat, W, hmax_ref, hmin_ref, part_ref):
  nb = nbr[0][:, :C]
  F = feat[0]
  crep = jnp.broadcast_to(F[:, None, :], (NT, KNN, C)).reshape(NT * KNN, C)
  diff = nb - crep
  if dup:
    fe = jnp.concatenate([diff, diff, crep, crep], axis=1)
  else:
    fe = jnp.concatenate([diff, crep], axis=1)
  h = lax.dot_general(fe, W[...], (((1,), (1,)), ((), ())),
                      preferred_element_type=jnp.float32)
  O = h.shape[1]
  h3 = h.reshape(NT, KNN, O)
  hmax_ref[0] = jnp.max(h3, axis=1)
  hmin_ref[0] = jnp.min(h3, axis=1)
  part_ref[0, 0, 0] = jnp.sum(h, axis=0)
  part_ref[0, 0, 1] = jnp.sum(h * h, axis=0)


def _conv(nbr_g, feat, W, O, C_pad, dup=False, interpret=False):
  C = feat.shape[-1]
  return pl.pallas_call(
      functools.partial(_conv_body, C, dup),
      grid=(B, NB),
      in_specs=[
          pl.BlockSpec((1, NT * KNN, C_pad), lambda b, t: (b, t, 0)),
          pl.BlockSpec((1, NT, C), lambda b, t: (b, t, 0)),
          _full_spec(W.shape),
      ],
      out_specs=[
          pl.BlockSpec((1, NT, O), lambda b, t: (b, t, 0)),
          pl.BlockSpec((1, NT, O), lambda b, t: (b, t, 0)),
          pl.BlockSpec((1, 1, 2, O), lambda b, t: (b, t, 0, 0)),
      ],
      out_shape=[
          jax.ShapeDtypeStruct((B, N, O), jnp.float32),
          jax.ShapeDtypeStruct((B, N, O), jnp.float32),
          jax.ShapeDtypeStruct((B, NB, 2, O), jnp.float32),
      ],
      interpret=interpret,
  )(nbr_g, feat, W)


def _combine_body(hmax, hmin, part, gam, bet, out_ref):
  s = jnp.sum(part[...], axis=(0, 1))
  T = B * N * KNN
  mean = s[0] / T
  var = s[1] / T - mean * mean
  a = gam[...] / jnp.sqrt(var + EPS)
  sel = jnp.where((a >= 0.0)[None, :], hmax[0], hmin[0])
  t = a[None, :] * (sel - mean[None, :]) + bet[...][None, :]
  out_ref[0] = jnp.where(t >= 0.0, t, 0.2 * t)


def _combine(hmax, hmin, part, gam, bet, O, interpret=False):
  return pl.pallas_call(
      _combine_body,
      grid=(B,),
      in_specs=[
          pl.BlockSpec((1, N, O), lambda b: (b, 0, 0)),
          pl.BlockSpec((1, N, O), lambda b: (b, 0, 0)),
          _full_spec((B, NB, 2, O)),
          _full_spec((O,)),
          _full_spec((O,)),
      ],
      out_specs=pl.BlockSpec((1, N, O), lambda b: (b, 0, 0)),
      out_shape=jax.ShapeDtypeStruct((B, N, O), jnp.float32),
      interpret=interpret,
  )(hmax, hmin, part, gam, bet)


def _head_body(u, W, mx_ref, mn_ref, s_ref, s2_ref):
  V = lax.dot_general(u[0], W[...], (((1,), (1,)), ((), ())),
                      preferred_element_type=jnp.float32)
  mx_ref[0, 0] = jnp.max(V, axis=0)
  mn_ref[0, 0] = jnp.min(V, axis=0)
  s_ref[0, 0] = jnp.sum(V, axis=0)
  s2_ref[0, 0] = jnp.sum(V * V, axis=0)


def _head(u, W, interpret=False):
  C = u.shape[-1]
  O = W.shape[0]
  return pl.pallas_call(
      _head_body,
      grid=(B,),
      in_specs=[
          pl.BlockSpec((1, N, C), lambda b: (b, 0, 0)),
          _full_spec(W.shape),
      ],
      out_specs=[pl.BlockSpec((1, 1, O), lambda b: (b, 0, 0))] * 4,
      out_shape=[jax.ShapeDtypeStruct((B, 1, O), jnp.float32)] * 4,
      interpret=interpret,
  )(u, W)


def _lrelu(t):
  return jnp.where(t >= 0.0, t, 0.2 * t)


def _mlp_body(h1m, h1n, h1s, h1s2, h2m, h2n, h2s, h2s2,
              g2m, b2m, g5, b5, L1, g6, b6, L2, bl2, g7, b7, L3, bl3,
              out_ref):
  T = B * N

  def pool(hm, hn, hs, hs2, g, bb):
    mean = jnp.sum(hs[...], axis=0) / T
    var = jnp.sum(hs2[...], axis=0) / T - mean * mean
    a = g[...] / jnp.sqrt(var + EPS)
    sel = jnp.where((a >= 0.0)[None, :], hm[...], hn[...])
    return _lrelu(a[None, :] * (sel - mean[None, :]) + bb[...][None, :])

  q1 = pool(h1m, h1n, h1s, h1s2, g2m, b2m)
  q2 = pool(h2m, h2n, h2s, h2s2, g5, b5)
  zc = jnp.concatenate([q1, q2], axis=1)

  def bn0(t, g, bb):
    m = jnp.mean(t, axis=0, keepdims=True)
    v = jnp.mean(t * t, axis=0, keepdims=True) - m * m
    return (t - m) / jnp.sqrt(v + EPS) * g[...][None, :] + bb[...][None, :]

  t = lax.dot_general(zc, L1[...], (((1,), (1,)), ((), ())),
                      preferred_element_type=jnp.float32)
  t = _lrelu(bn0(t, g6, b6))
  t = lax.dot_general(t, L2[...], (((1,), (1,)), ((), ())),
                      preferred_element_type=jnp.float32) + bl2[...][None, :]
  t = _lrelu(bn0(t, g7, b7))
  out_ref[...] = lax.dot_general(t, L3[...], (((1,), (1,)), ((), ())),
                                 preferred_element_type=jnp.float32) \
      + bl3[...][None, :]


def _mlp(args, interpret=False):
  return pl.pallas_call(
      _mlp_body,
      in_specs=[_full_spec(a.shape) for a in args],
      out_specs=_full_spec((B, 40)),
      out_shape=jax.ShapeDtypeStruct((B, 40), jnp.float32),
      interpret=interpret,
  )(*args)


def _edge_stage(feat, dfeat, W, gam, bet, O, dup=False, interpret=False):
  C = feat.shape[-1]
  C_pad = max(C, 128)
  idx = _front(dfeat, interpret=interpret)
  fp = feat.reshape(P, C)
  if C_pad > C:
    fp = jnp.pad(fp, ((0, 0), (0, C_pad - C)))
  nbr = _sc_gather(fp, idx.reshape(P * KNN), C_pad)
  nbr = nbr.reshape(B, N * KNN, C_pad)
  hmax, hmin, part = _conv(nbr, feat, W, O, C_pad, dup=dup, interpret=interpret)
  return _combine(hmax, hmin, part, gam, bet, O, interpret=interpret)


def kernel(x, W1, g1, b1, W2, g2, b2, W2m, g2m, b2m, W3, g3, b3, W4, g4, b4,
           W5, g5, b5, L1, g6, b6, L2, bl2, g7, b7, L3, bl3):
  xp = jnp.transpose(x, (0, 2, 1))
  x1 = _edge_stage(xp, xp, W1, g1, b1, 64)
  x2 = _edge_stage(x1, x1, W2, g2, b2, 64)
  u = jnp.concatenate([x1, x2], axis=-1)
  ud = jnp.concatenate([u, u], axis=-1)
  x3 = _edge_stage(u, ud, W3, g3, b3, 256, dup=True)
  x4 = _edge_stage(x3, x3, W4, g4, b4, 256)
  xc = jnp.concatenate([x3, x4], axis=-1)
  h1 = [t.reshape(B, -1) for t in _head(u, W2m)]
  h2 = [t.reshape(B, -1) for t in _head(xc, W5)]
  logits = _mlp([h1[0], h1[1], h1[2], h1[3], h2[0], h2[1], h2[2], h2[3],
                 g2m, b2m, g5, b5, L1, g6, b6, L2, bl2, g7, b7, L3, bl3])
  return logits, x, x
